# Initial kernel scaffold; baseline (speedup 1.0000x reference)
import jax
import jax.numpy as jnp
from jax.experimental import pallas as pl


def kernel(features):
    raise NotImplementedError("write your pallas kernel here")


# speedup vs baseline: 7.8778x; 7.8778x over previous
import jax
import jax.numpy as jnp
from jax.experimental import pallas as pl

TOP_K = 8
THRESHOLD = 1e-4
INV_TEMPERATURE = 10.0


def _graph_block_kernel(x_ref, o_ref):
    x = jnp.transpose(x_ref[0], (1, 0, 2))
    n = x.shape[1]
    norm = jnp.sqrt(jnp.sum(x * x, axis=-1, keepdims=True))
    xn = x / jnp.maximum(norm, 1e-12)
    adj = jax.lax.dot_general(
        xn, xn, (((2,), (2,)), ((0,), (0,))),
        preferred_element_type=jnp.float32,
    )
    adj = adj * INV_TEMPERATURE
    m = jnp.max(adj, axis=-1, keepdims=True)
    e = jnp.exp(adj - m)
    a = e / jnp.sum(e, axis=-1, keepdims=True)

    col = jax.lax.broadcasted_iota(jnp.int32, a.shape, 2)
    work = a
    keep = jnp.zeros(a.shape, jnp.bool_)
    for _ in range(TOP_K):
        mx = jnp.max(work, axis=-1, keepdims=True)
        idx = jnp.where(work == mx, col, n)
        sel = col == jnp.min(idx, axis=-1, keepdims=True)
        keep = jnp.logical_or(keep, sel)
        work = jnp.where(sel, -jnp.inf, work)

    a = jnp.where(keep & (a > THRESHOLD), a, 0.0)
    o_ref[0] = (a + jnp.transpose(a, (0, 2, 1))) * 0.5


def kernel(features):
    B, N, T, D = features.shape
    Tt = 8
    return pl.pallas_call(
        _graph_block_kernel,
        grid=(B, T // Tt),
        in_specs=[pl.BlockSpec((1, N, Tt, D), lambda b, t: (b, 0, t, 0))],
        out_specs=pl.BlockSpec((1, Tt, N, N), lambda b, t: (b, t, 0, 0)),
        out_shape=jax.ShapeDtypeStruct((B, T, N, N), jnp.float32),
    )(features)


# speedup vs baseline: 17.4929x; 2.2205x over previous
import jax
import jax.numpy as jnp
from jax.experimental import pallas as pl

TOP_K = 8
THRESHOLD = 1e-4
INV_TEMPERATURE = 10.0


def _graph_block_kernel(x_ref, o_ref):
    x = jnp.transpose(x_ref[0], (1, 0, 2))
    norm2 = jnp.sum(x * x, axis=-1, keepdims=True)
    xn = x * jax.lax.rsqrt(jnp.maximum(norm2, 1e-24))
    adj = jax.lax.dot_general(
        xn, xn, (((2,), (2,)), ((0,), (0,))),
        preferred_element_type=jnp.float32,
    )

    m = jnp.max(adj, axis=-1, keepdims=True)
    e = jnp.exp((adj - m) * INV_TEMPERATURE)
    a = e / jnp.sum(e, axis=-1, keepdims=True)

    work = adj
    for _ in range(TOP_K - 1):
        mx = jnp.max(work, axis=-1, keepdims=True)
        work = jnp.where(work < mx, work, -jnp.inf)
    t8 = jnp.max(work, axis=-1, keepdims=True)

    a = jnp.where((adj >= t8) & (a > THRESHOLD), a, 0.0)
    o_ref[0] = (a + jnp.transpose(a, (0, 2, 1))) * 0.5


def kernel(features):
    B, N, T, D = features.shape
    Tt = 8
    return pl.pallas_call(
        _graph_block_kernel,
        grid=(B, T // Tt),
        in_specs=[pl.BlockSpec((1, N, Tt, D), lambda b, t: (b, 0, t, 0))],
        out_specs=pl.BlockSpec((1, Tt, N, N), lambda b, t: (b, t, 0, 0)),
        out_shape=jax.ShapeDtypeStruct((B, T, N, N), jnp.float32),
    )(features)


# speedup vs baseline: 22.0422x; 1.2601x over previous
import jax
import jax.numpy as jnp
from jax.experimental import pallas as pl

TOP_K = 8
THRESHOLD = 1e-4
INV_TEMPERATURE = 10.0


def _graph_block_kernel(x_ref, o_ref):
    x = jnp.transpose(x_ref[0], (1, 0, 2))
    norm2 = jnp.sum(x * x, axis=-1, keepdims=True)
    xn = x * jax.lax.rsqrt(jnp.maximum(norm2, 1e-24))
    adj = jax.lax.dot_general(
        xn, xn, (((2,), (2,)), ((0,), (0,))),
        preferred_element_type=jnp.float32,
    )

    e = jnp.exp((adj - 1.0) * INV_TEMPERATURE)
    s = jnp.sum(e, axis=-1, keepdims=True)
    r = 1.0 / s

    row = jax.lax.broadcasted_iota(jnp.int32, adj.shape, 1)
    col = jax.lax.broadcasted_iota(jnp.int32, adj.shape, 2)
    work = jnp.where(row == col, -jnp.inf, adj)
    for _ in range(TOP_K - 2):
        mx = jnp.max(work, axis=-1, keepdims=True)
        work = jnp.where(work < mx, work, -jnp.inf)
    t8 = jnp.max(work, axis=-1, keepdims=True)

    keep = (adj >= t8) & (e > THRESHOLD * s)
    a = jnp.where(keep, e, 0.0) * r
    o_ref[0] = (a + jnp.transpose(a, (0, 2, 1))) * 0.5


def kernel(features):
    B, N, T, D = features.shape
    Tt = 16
    return pl.pallas_call(
        _graph_block_kernel,
        grid=(B, T // Tt),
        in_specs=[pl.BlockSpec((1, N, Tt, D), lambda b, t: (b, 0, t, 0))],
        out_specs=pl.BlockSpec((1, Tt, N, N), lambda b, t: (b, t, 0, 0)),
        out_shape=jax.ShapeDtypeStruct((B, T, N, N), jnp.float32),
    )(features)


# speedup vs baseline: 22.5788x; 1.0243x over previous
import jax
import jax.numpy as jnp
from jax.experimental import pallas as pl

TOP_K = 8
THRESHOLD = 1e-4
INV_TEMPERATURE = 10.0


def _graph_block_kernel(x_ref, o_ref):
    x = jnp.transpose(x_ref[0], (1, 0, 2))
    norm2 = jnp.sum(x * x, axis=-1, keepdims=True)
    xn = x * jax.lax.rsqrt(jnp.maximum(norm2, 1e-24))
    adj = jax.lax.dot_general(
        xn, xn, (((2,), (2,)), ((0,), (0,))),
        preferred_element_type=jnp.float32,
    )

    e = jnp.exp((adj - 1.0) * INV_TEMPERATURE)
    s = jnp.sum(e, axis=-1, keepdims=True)
    r = 1.0 / s

    row = jax.lax.broadcasted_iota(jnp.int32, adj.shape, 1)
    col = jax.lax.broadcasted_iota(jnp.int32, adj.shape, 2)
    work = jnp.where(row == col, -jnp.inf, adj)
    for _ in range(TOP_K - 2):
        mx = jnp.max(work, axis=-1, keepdims=True)
        work = jnp.where(work < mx, work, -jnp.inf)
    t8 = jnp.max(work, axis=-1, keepdims=True)

    keep = (adj >= t8) & (e > THRESHOLD * s)
    a = jnp.where(keep, e, 0.0) * r
    o_ref[0] = (a + jnp.transpose(a, (0, 2, 1))) * 0.5


def kernel(features):
    B, N, T, D = features.shape
    Tt = 64
    return pl.pallas_call(
        _graph_block_kernel,
        grid=(B, T // Tt),
        in_specs=[pl.BlockSpec((1, N, Tt, D), lambda b, t: (b, 0, t, 0))],
        out_specs=pl.BlockSpec((1, Tt, N, N), lambda b, t: (b, t, 0, 0)),
        out_shape=jax.ShapeDtypeStruct((B, T, N, N), jnp.float32),
    )(features)
